# baseline (speedup 1.0000x reference)
import functools

import jax
import jax.numpy as jnp
from jax import lax
from jax.experimental import pallas as pl
from jax.experimental.pallas import tpu as pltpu
from jax.experimental.pallas import tpu_sc as plsc

NC = 2
NS = 16
NW = NC * NS


def _row_dma_gather(tbl_hbm, idx_v, out_hbm, buf, sem, sem_wb, base, n):
    @pl.loop(0, n // 16)
    def _(g):
        k0 = g * 16
        iv = idx_v[pl.ds(k0, 16)]
        for j in range(16):
            pltpu.async_copy(tbl_hbm.at[iv[j]], buf.at[k0 + j], sem)

    pltpu.make_async_copy(tbl_hbm.at[pl.ds(0, n)], buf, sem).wait()
    pltpu.async_copy(buf, out_hbm.at[pl.ds(base, n)], sem_wb).wait()


def _sc_gather_pat(pat_lat, idx_pat):
    B = idx_pat.shape[0]
    D = pat_lat.shape[1]
    b_per_w = B // NW
    mesh = plsc.VectorSubcoreMesh(core_axis_name="c", subcore_axis_name="s")

    @functools.partial(
        pl.kernel,
        mesh=mesh,
        out_type=jax.ShapeDtypeStruct((B, D), jnp.float32),
        scratch_types=[
            pltpu.VMEM((b_per_w,), jnp.int32),
            pltpu.VMEM((b_per_w, 32), jnp.float32),
            pltpu.SemaphoreType.DMA,
            pltpu.SemaphoreType.DMA,
        ],
    )
    def gather_kernel(pat_hbm, ip_hbm, pat_out, ipv, buf, sem, sem_wb):
        wid = lax.axis_index("s") * NC + lax.axis_index("c")
        base = wid * b_per_w
        pltpu.sync_copy(ip_hbm.at[pl.ds(base, b_per_w)], ipv)
        _row_dma_gather(pat_hbm, ipv, pat_out, buf, sem, sem_wb, base,
                        b_per_w)

    return gather_kernel(pat_lat, idx_pat)


def _sc_gather_small(meas_lat, time_lat, idx_meas, idx_t):
    B = idx_meas.shape[0]
    D = meas_lat.shape[1]
    b_per_w = B // NW
    out_t = jax.ShapeDtypeStruct((B, D), jnp.float32)
    mesh = plsc.VectorSubcoreMesh(core_axis_name="c", subcore_axis_name="s")

    @functools.partial(
        pl.kernel,
        mesh=mesh,
        out_type=(out_t, out_t),
        scratch_types=[
            pltpu.VMEM((b_per_w,), jnp.int32),
            pltpu.VMEM((b_per_w,), jnp.int32),
            pltpu.VMEM((b_per_w, 32), jnp.float32),
            pltpu.SemaphoreType.DMA,
            pltpu.SemaphoreType.DMA,
        ],
    )
    def gather_kernel(meas_hbm, tim_hbm, im_hbm, it_hbm,
                      meas_out, tim_out, imv, itv, buf, sem, sem_wb):
        wid = lax.axis_index("s") * NC + lax.axis_index("c")
        base = wid * b_per_w
        pltpu.sync_copy(im_hbm.at[pl.ds(base, b_per_w)], imv)
        pltpu.sync_copy(it_hbm.at[pl.ds(base, b_per_w)], itv)
        _row_dma_gather(meas_hbm, imv, meas_out, buf, sem, sem_wb, base,
                        b_per_w)
        _row_dma_gather(tim_hbm, itv, tim_out, buf, sem, sem_wb, base,
                        b_per_w)

    return gather_kernel(meas_lat, time_lat, idx_meas, idx_t)


def _tc_combine_body(pat_ref, meas_ref, tim_ref, cu_ref, cw_ref,
                     bu_ref, bw_ref, s_ref, out_ref):
    u = jnp.dot(cu_ref[...], bu_ref[...],
                preferred_element_type=jnp.float32,
                precision=lax.Precision.HIGHEST)
    w = jnp.dot(cw_ref[...], bw_ref[...],
                preferred_element_type=jnp.float32,
                precision=lax.Precision.HIGHEST)
    prod = (pat_ref[...] + u) * meas_ref[...] * (tim_ref[...] + w)
    out_ref[...] = jnp.dot(prod, s_ref[...],
                           preferred_element_type=jnp.float32,
                           precision=lax.Precision.HIGHEST)


def kernel(idx_pat, idx_meas, idx_t, cov_u, cov_w, pat_lat, meas_lat,
           time_lat, beta_u, beta_w):
    B = idx_pat.shape[0]
    D = pat_lat.shape[1]
    NU = cov_u.shape[1]
    NWc = cov_w.shape[1]
    PACK = 128 // D
    R = B // PACK

    meas_g, tim_g = _sc_gather_small(
        meas_lat, time_lat,
        idx_meas.astype(jnp.int32), idx_t.astype(jnp.int32))
    pat_g = _sc_gather_pat(pat_lat, idx_pat.astype(jnp.int32))

    pat4 = pat_g.reshape(R, PACK * D)
    meas4 = meas_g.reshape(R, PACK * D)
    tim4 = tim_g.reshape(R, PACK * D)
    cu4 = cov_u.reshape(R, PACK * NU)
    cw4 = cov_w.reshape(R, PACK * NWc)

    eye = jnp.eye(PACK, dtype=jnp.float32)
    bu_bd = jnp.kron(eye, beta_u)
    bw_bd = jnp.kron(eye, beta_w)
    s_mat = jnp.kron(eye, jnp.ones((D, 1), jnp.float32))

    out4 = pl.pallas_call(
        _tc_combine_body,
        out_shape=jax.ShapeDtypeStruct((R, PACK), jnp.float32),
    )(pat4, meas4, tim4, cu4, cw4, bu_bd, bw_bd, s_mat)
    return out4.reshape(B)
